# baseline (speedup 1.0000x reference)
import functools

import jax
import jax.numpy as jnp
from jax import lax
from jax.experimental import pallas as pl
from jax.experimental.pallas import tpu as pltpu
from jax.experimental.pallas import tpu_sc as plsc

VOCAB_SZ = 1000000
VOCAB_MAIN = 999936
D_MODEL = 64
BATCH = 4096
SEQ = 200
N_TOK = BATCH * SEQ
NUM_CORES = 2
NUM_SUBCORES = 16
NW = NUM_CORES * NUM_SUBCORES

_mesh = plsc.VectorSubcoreMesh(core_axis_name="c", subcore_axis_name="s")
_iota16 = None

KA = 256
NCA = VOCAB_MAIN // KA
PER_WA = NCA // NW
REM_A = NCA - PER_WA * NW


@functools.partial(
    pl.kernel,
    mesh=_mesh,
    out_type=jax.ShapeDtypeStruct((VOCAB_SZ // 2, 128), jnp.float32),
    scratch_types=[
        pltpu.VMEM((2, 8, 2, 8, 128), jnp.float32),
        pltpu.VMEM((2, KA // 2, 128), jnp.float32),
        pltpu.SemaphoreType.DMA((2,)),
        pltpu.SemaphoreType.DMA((2,)),
    ],
    compiler_params=pltpu.CompilerParams(
        use_tc_tiling_on_sc=True, needs_layout_passes=False
    ),
)
def _transpose_sc(tblT_hbm, tailp_hbm, out_hbm, ibuf, obuf, isem, osem):
    wid = lax.axis_index("s") * NUM_CORES + lax.axis_index("c")
    iota = jnp.arange(16, dtype=jnp.int32)
    zeros = jnp.zeros((16,), jnp.int32)
    f_pat = [
        ((16 * m + iota) // 8) * 2048 + ((16 * m + iota) % 8) * 128
        for m in range(4)
    ]

    def fire_in(c, b):
        c0 = c * KA
        for db in range(8):
            for cb in range(2):
                pltpu.async_copy(
                    tblT_hbm.at[pl.ds(db * 8, 8), pl.ds(c0 + cb * 128, 128)],
                    ibuf.at[b, db, cb],
                    isem.at[b],
                )

    def wait_in(c, b):
        c0 = c * KA
        for db in range(8):
            for cb in range(2):
                pltpu.make_async_copy(
                    tblT_hbm.at[pl.ds(db * 8, 8), pl.ds(c0 + cb * 128, 128)],
                    ibuf.at[b, db, cb],
                    isem.at[b],
                ).wait()

    def fire_out(c, b):
        pltpu.async_copy(
            obuf.at[b], out_hbm.at[pl.ds(c * (KA // 2), KA // 2)], osem.at[b]
        )

    def wait_out(c, b):
        pltpu.make_async_copy(
            obuf.at[b], out_hbm.at[pl.ds(c * (KA // 2), KA // 2)], osem.at[b]
        ).wait()

    def transpose(b):
        @plsc.parallel_loop(0, KA, 4, carry=jnp.int32(0))
        def _loop(tk, carry):
            vs = []
            for u in range(4):
                tokl = tk + u
                off = (tokl // 128) * 1024 + tokl % 128
                off_s = jnp.full((16,), off, jnp.int32)
                for m in range(4):
                    vs.append(
                        plsc.load_gather(
                            ibuf.at[b], [zeros, zeros, zeros, f_pat[m] + off_s]
                        )
                    )
            for u in range(4):
                tokl = tk + u
                for m in range(4):
                    obuf[
                        b, tokl // 2, pl.ds((tokl % 2) * 64 + m * 16, 16)
                    ] = vs[u * 4 + m]
            return carry

    def process(c, b, k, last_k):
        @pl.when(k + 1 <= last_k)
        def _():
            fire_in(c + 1, b ^ 1)

        wait_in(c, b)

        @pl.when(k >= 2)
        def _():
            wait_out(c - 2, b)

        transpose(b)
        fire_out(c, b)

    base = wid * PER_WA
    fire_in(base, 0)

    def pair_body(g, carry):
        process(base + 2 * g, 0, 2 * g, PER_WA - 1)
        process(base + 2 * g + 1, 1, 2 * g + 1, PER_WA - 1)
        return carry

    lax.fori_loop(0, PER_WA // 2, pair_body, 0)
    wait_out(base + PER_WA - 2, 0)
    wait_out(base + PER_WA - 1, 1)

    @pl.when(wid < REM_A)
    def _():
        c = NW * PER_WA + wid
        fire_in(c, 0)
        wait_in(c, 0)
        transpose(0)
        fire_out(c, 0)
        wait_out(c, 0)

    @pl.when(wid == REM_A)
    def _():
        pltpu.sync_copy(tailp_hbm, obuf.at[0, pl.ds(0, 32)])
        pltpu.sync_copy(
            obuf.at[0, pl.ds(0, 32)],
            out_hbm.at[pl.ds(VOCAB_MAIN // 2, 32)],
        )


NB = 2
CT = NB * 128
CH_PER_T = BATCH // CT
NCB = SEQ * CH_PER_T
PER_WB = NCB // NW


@functools.partial(
    pl.kernel,
    mesh=_mesh,
    out_type=jax.ShapeDtypeStruct((SEQ, 8, BATCH // 128, 8, 128), jnp.float32),
    scratch_types=[
        pltpu.VMEM((2, CT), jnp.int32),
        pltpu.VMEM((2, CT, D_MODEL), jnp.float32),
        pltpu.VMEM((2, 8, NB, 8, 128), jnp.float32),
        pltpu.SemaphoreType.DMA((2,)),
        pltpu.SemaphoreType.DMA((2,)),
    ],
    compiler_params=pltpu.CompilerParams(
        use_tc_tiling_on_sc=False, needs_layout_passes=False
    ),
)
def _gather_fmt(idx_hbm, tbl_hbm, out_hbm, idx_v, rows_v, t_v, gsem, osem):
    wid = lax.axis_index("s") * NUM_CORES + lax.axis_index("c")
    iota = jnp.arange(16, dtype=jnp.int32)
    zeros = jnp.zeros((16,), jnp.int32)
    rpat = [
        (bb * 128 + q * 16 + iota) * D_MODEL
        for bb in range(NB)
        for q in range(8)
    ]

    def fire_gather(c, b):
        base = c * CT
        pltpu.sync_copy(idx_hbm.at[pl.ds(base, CT)], idx_v.at[b])
        pltpu.async_copy(tbl_hbm.at[idx_v.at[b]], rows_v.at[b], gsem.at[b])

    def wait_gather(b):
        pltpu.make_async_copy(
            tbl_hbm.at[idx_v.at[b]], rows_v.at[b], gsem.at[b]
        ).wait()

    def fire_out(c, b):
        t = c // CH_PER_T
        bb0 = (c % CH_PER_T) * NB
        for db in range(8):
            pltpu.async_copy(
                t_v.at[b, db], out_hbm.at[t, db, pl.ds(bb0, NB)], osem.at[b]
            )

    def wait_out(c, b):
        t = c // CH_PER_T
        bb0 = (c % CH_PER_T) * NB
        for db in range(8):
            pltpu.make_async_copy(
                t_v.at[b, db], out_hbm.at[t, db, pl.ds(bb0, NB)], osem.at[b]
            ).wait()

    def transpose(b):
        @plsc.parallel_loop(0, 8, 1, carry=jnp.int32(0))
        def _loop(db, carry):
            for ds in range(8):
                col = jnp.full((16,), db * 8 + ds, jnp.int32)
                vs = [
                    plsc.load_gather(rows_v.at[b], [zeros, rpat[k] + col])
                    for k in range(NB * 8)
                ]
                for bb in range(NB):
                    for q in range(8):
                        t_v[b, db, bb, ds, pl.ds(q * 16, 16)] = vs[bb * 8 + q]
            return carry

    def process(c, b, k):
        @pl.when(k + 1 <= PER_WB - 1)
        def _():
            fire_gather(c + 1, b ^ 1)

        wait_gather(b)

        @pl.when(k >= 2)
        def _():
            wait_out(c - 2, b)

        transpose(b)
        fire_out(c, b)

    base = wid * PER_WB
    fire_gather(base, 0)

    def pair_body(g, carry):
        process(base + 2 * g, 0, 2 * g)
        process(base + 2 * g + 1, 1, 2 * g + 1)
        return carry

    lax.fori_loop(0, PER_WB // 2, pair_body, 0)
    wait_out(base + PER_WB - 2, 0)
    wait_out(base + PER_WB - 1, 1)


def kernel(tok_ids, table):
    idx_flat = tok_ids.T.reshape(-1)
    tailp = table[VOCAB_MAIN:].reshape(32, 128)
    packed = _transpose_sc(table.T, tailp)
    tbl_rm = packed.reshape(VOCAB_SZ, D_MODEL)
    out5 = _gather_fmt(idx_flat, tbl_rm)
    return out5.transpose(2, 4, 0, 1, 3).reshape(BATCH, SEQ, D_MODEL)


# speedup vs baseline: 1.6703x; 1.6703x over previous
import functools

import jax
import jax.numpy as jnp
from jax import lax
from jax.experimental import pallas as pl
from jax.experimental.pallas import tpu as pltpu
from jax.experimental.pallas import tpu_sc as plsc

VOCAB_SZ = 1000000
D_MODEL = 64
BATCH = 4096
SEQ = 200
N_TOK = BATCH * SEQ
NUM_CORES = 2
NUM_SUBCORES = 16
NW = NUM_CORES * NUM_SUBCORES
CHUNK = 512
PER_W = N_TOK // (NW * CHUNK)


@functools.partial(
    pl.kernel,
    mesh=plsc.VectorSubcoreMesh(core_axis_name="c", subcore_axis_name="s"),
    out_type=jax.ShapeDtypeStruct((N_TOK, D_MODEL), jnp.float32),
    scratch_types=[
        pltpu.VMEM((2, CHUNK), jnp.int32),
        pltpu.VMEM((2, CHUNK, D_MODEL), jnp.float32),
        pltpu.SemaphoreType.DMA((2,)),
        pltpu.SemaphoreType.DMA((2,)),
    ],
    compiler_params=pltpu.CompilerParams(use_tc_tiling_on_sc=False),
)
def _gather_sc(idx_hbm, tbl_hbm, out_hbm, idx_v, rows_v, gsem, osem):
    wid = lax.axis_index("s") * NUM_CORES + lax.axis_index("c")

    def fire_gather(k, b):
        base = (wid * PER_W + k) * CHUNK
        pltpu.sync_copy(idx_hbm.at[pl.ds(base, CHUNK)], idx_v.at[b])
        pltpu.async_copy(tbl_hbm.at[idx_v.at[b]], rows_v.at[b], gsem.at[b])

    def wait_gather(b):
        pltpu.make_async_copy(
            tbl_hbm.at[idx_v.at[b]], rows_v.at[b], gsem.at[b]
        ).wait()

    def fire_out(k, b):
        base = (wid * PER_W + k) * CHUNK
        pltpu.async_copy(
            rows_v.at[b], out_hbm.at[pl.ds(base, CHUNK)], osem.at[b]
        )

    def wait_out(k, b):
        base = (wid * PER_W + k) * CHUNK
        pltpu.make_async_copy(
            rows_v.at[b], out_hbm.at[pl.ds(base, CHUNK)], osem.at[b]
        ).wait()

    def process(k, b):
        @pl.when(k + 1 <= PER_W - 1)
        def _():
            fire_gather(k + 1, b ^ 1)

        wait_gather(b)

        @pl.when(k >= 2)
        def _():
            wait_out(k - 2, b)

        fire_out(k, b)

    fire_gather(0, 0)

    def pair_body(g, carry):
        process(2 * g, 0)
        process(2 * g + 1, 1)
        return carry

    lax.fori_loop(0, PER_W // 2, pair_body, 0)
    wait_out(PER_W - 2, 0)
    wait_out(PER_W - 1, 1)


def kernel(tok_ids, table):
    idx_flat = tok_ids.reshape(-1)
    out = _gather_sc(idx_flat, table)
    return out.reshape(BATCH, SEQ, D_MODEL)
